# baseline (speedup 1.0000x reference)
import functools

import jax
import jax.numpy as jnp
from jax import lax
from jax.experimental import pallas as pl
from jax.experimental.pallas import tpu as pltpu
from jax.experimental.pallas import tpu_sc as plsc

NC = 2
NS = 16
NW = NC * NS
CH = 64
GRP = 8
LANES = 16


def _proj_body(x_ref, w_in_ref, b_in_ref, w1_ref, w2_ref, bm_ref, a_ref, b_ref):
    h = jnp.dot(x_ref[...], w_in_ref[...], preferred_element_type=jnp.float32)
    h = jnp.maximum(h + b_in_ref[...], 0.0)
    a_ref[...] = jnp.dot(h, w1_ref[...], preferred_element_type=jnp.float32)
    b_ref[...] = (
        jnp.dot(h, w2_ref[...], preferred_element_type=jnp.float32) + bm_ref[...]
    )


@functools.partial(jax.jit, static_argnames=("blk",))
def _proj(x, w_in, b_in, w1, w2, bm, blk=1000):
    n, d = x.shape
    hid = w_in.shape[1]
    grid = n // blk
    return pl.pallas_call(
        _proj_body,
        grid=(grid,),
        in_specs=[
            pl.BlockSpec((blk, d), lambda i: (i, 0)),
            pl.BlockSpec((d, hid), lambda i: (0, 0)),
            pl.BlockSpec((1, hid), lambda i: (0, 0)),
            pl.BlockSpec((hid, hid), lambda i: (0, 0)),
            pl.BlockSpec((hid, hid), lambda i: (0, 0)),
            pl.BlockSpec((1, hid), lambda i: (0, 0)),
        ],
        out_specs=[
            pl.BlockSpec((blk, hid), lambda i: (i, 0)),
            pl.BlockSpec((blk, hid), lambda i: (i, 0)),
        ],
        out_shape=[
            jax.ShapeDtypeStruct((n, hid), jnp.float32),
            jax.ShapeDtypeStruct((n, hid), jnp.float32),
        ],
    )(x, w_in, b_in.reshape(1, hid), w1, w2, bm.reshape(1, hid))


def _make_sc(nch0, nch1, npad, hid):
    assert nch0 % (2 * GRP) == 0 and nch1 % (2 * GRP) == 0
    rows_per_tile = npad // NS
    assert rows_per_tile % CH == 0
    vpr = hid // LANES

    mesh = plsc.VectorSubcoreMesh(core_axis_name="c", subcore_axis_name="s")

    @functools.partial(
        pl.kernel,
        out_type=jax.ShapeDtypeStruct((NC, npad, hid), jnp.float32),
        mesh=mesh,
        scratch_types=[
            [pltpu.VMEM((GRP, 2 * CH), jnp.int32)] * 2,
            [pltpu.VMEM((GRP, CH), jnp.int32)] * 2,
            [pltpu.VMEM((2 * CH, hid), jnp.float32)] * 2,
            pltpu.VMEM((CH, hid), jnp.float32),
            pltpu.VMEM_SHARED((npad, hid), jnp.float32),
            [pltpu.SemaphoreType.DMA] * 2,
            [pltpu.SemaphoreType.DMA] * 2,
        ],
    )
    def sc_edges(t_hbm, t2_hbm, gidx_hbm, darr_hbm, out_hbm,
                 gi, da, abuf, mbuf, acc, sem_i, sem_g):
        cid = lax.axis_index("c")
        sid = lax.axis_index("s")
        chunk0 = jnp.where(cid == 0, sid * nch0, NS * nch0 + sid * nch1)
        ngroups = jnp.where(cid == 0, nch0 // GRP, nch1 // GRP)

        def issue_idx(gp, g):
            row0 = chunk0 + g * GRP
            pltpu.async_copy(gidx_hbm.at[pl.ds(row0, GRP)], gi[gp], sem_i[gp])
            pltpu.async_copy(darr_hbm.at[pl.ds(row0, GRP)], da[gp], sem_i[gp])

        def wait_idx(gp):
            pltpu.make_async_copy(
                gidx_hbm.at[pl.ds(chunk0, GRP)], gi[gp], sem_i[gp]).wait()
            pltpu.make_async_copy(
                darr_hbm.at[pl.ds(chunk0, GRP)], da[gp], sem_i[gp]).wait()

        def issue_gather(p, gp, j):
            @pl.when(cid == 0)
            def _():
                pltpu.async_copy(t_hbm.at[gi[gp].at[j]], abuf[p], sem_g[p])

            @pl.when(cid != 0)
            def _():
                pltpu.async_copy(t2_hbm.at[gi[gp].at[j]], abuf[p], sem_g[p])

        def wait_gather(p, gp, j):
            @pl.when(cid == 0)
            def _():
                pltpu.make_async_copy(
                    t_hbm.at[gi[gp].at[j]], abuf[p], sem_g[p]).wait()

            @pl.when(cid != 0)
            def _():
                pltpu.make_async_copy(
                    t2_hbm.at[gi[gp].at[j]], abuf[p], sem_g[p]).wait()

        scope_zero = jax.named_scope("phase_zero")
        scope_zero.__enter__()
        zero = jnp.zeros((LANES,), jnp.float32)

        def _zero_row(r, _):
            for c in range(vpr):
                mbuf[r, pl.ds(c * LANES, LANES)] = zero
            return 0

        lax.fori_loop(0, CH, _zero_row, 0)

        def _zero_acc(i, _):
            pltpu.sync_copy(mbuf, acc.at[pl.ds(sid * rows_per_tile + i * CH, CH)])
            return 0

        lax.fori_loop(0, rows_per_tile // CH, _zero_acc, 0)
        plsc.subcore_barrier()
        scope_zero.__exit__(None, None, None)

        scope_loop = jax.named_scope("phase_loop")
        scope_loop.__enter__()
        issue_idx(0, 0)
        wait_idx(0)
        issue_gather(0, 0, 0)
        issue_gather(1, 0, 1)

        @pl.loop(0, ngroups, step=2)
        def _groups(gbase):
            for gg in range(2):
                g = gbase + gg
                gp = gg
                for j in range(GRP):
                    p = j & 1
                    if j == 0:
                        g_next = jnp.where(g + 1 >= ngroups, 0, g + 1)
                        issue_idx(gp ^ 1, g_next)
                    if j == GRP - 3:
                        wait_idx(gp ^ 1)
                    wait_gather(p, gp, j)

                    def _row(r, _, p=p):
                        for cc in range(vpr):
                            s = pl.ds(cc * LANES, LANES)
                            mbuf[r, s] = jnp.maximum(
                                abuf[p][r, s] + abuf[p][r + CH, s], 0.0)
                        return 0

                    lax.fori_loop(0, CH, _row, 0)
                    if j < GRP - 2:
                        issue_gather(p, gp, j + 2)
                    else:
                        issue_gather(p, gp ^ 1, j + 2 - GRP)
                    pltpu.sync_copy(mbuf, acc.at[da[gp].at[j]], add=True)

        wait_gather(0, 0, 0)
        wait_gather(1, 0, 1)
        plsc.subcore_barrier()
        scope_loop.__exit__(None, None, None)

        with jax.named_scope("phase_drain"):
            r0 = sid * rows_per_tile
            pltpu.sync_copy(acc.at[pl.ds(r0, rows_per_tile)],
                            out_hbm.at[cid, pl.ds(r0, rows_per_tile)])

    return sc_edges


def _merge_body(p_ref, o_ref):
    o_ref[...] = p_ref[0] + p_ref[1]


@functools.partial(jax.jit, static_argnames=("n", "blk"))
def _merge(partials, n, blk=1000):
    npad, hid = partials.shape[1], partials.shape[2]
    return pl.pallas_call(
        _merge_body,
        grid=(n // blk,),
        in_specs=[pl.BlockSpec((2, blk, hid), lambda i: (0, i, 0))],
        out_specs=pl.BlockSpec((blk, hid), lambda i: (i, 0)),
        out_shape=jax.ShapeDtypeStruct((n, hid), jnp.float32),
    )(partials)


def kernel(x, edge_index, W_in, b_in, W_msg, b_msg):
    n, d = x.shape
    hid = W_in.shape[1]
    e = edge_index.shape[1]

    a, b = _proj(x, W_in, b_in, W_msg[:hid], W_msg[hid:], b_msg)

    quantum = NW * CH * 2 * GRP
    e_pad = -(-e // quantum) * quantum
    src = edge_index[0]
    dst = edge_index[1]
    if e_pad > e:
        src = jnp.concatenate([src, jnp.zeros((e_pad - e,), jnp.int32)])
        dst = jnp.concatenate([dst, jnp.full((e_pad - e,), n, jnp.int32)])
    b_pad = jnp.concatenate([b, jnp.zeros((1, hid), jnp.float32)], axis=0)

    table = jnp.concatenate([a, b_pad], axis=0)
    table2 = jnp.concatenate([b_pad, a], axis=0)
    tot_chunks = e_pad // CH
    src_c = src.reshape(tot_chunks, CH)
    dst_c = dst.reshape(tot_chunks, CH)
    gidx0 = jnp.concatenate([src_c, dst_c + n], axis=1)
    gidx1 = jnp.concatenate([src_c + (n + 1), dst_c], axis=1)
    darr = dst_c

    npad = -(-(n + 1) // (NS * CH)) * (NS * CH)
    tot_pair = e_pad // (CH * NS)
    k0 = int(round(0.5 * tot_pair / (2 * GRP))) * (2 * GRP)
    k1 = tot_pair - k0
    row_is_core1 = (jnp.arange(tot_chunks) >= NS * k0)[:, None]
    gidx = jnp.where(row_is_core1, gidx1, gidx0)
    sc_edges = _make_sc(k0, k1, npad, hid)
    partials = sc_edges(table, table2, gidx, darr)

    return _merge(partials, n)


# speedup vs baseline: 1.2124x; 1.2124x over previous
import functools

import jax
import jax.numpy as jnp
from jax import lax
from jax.experimental import pallas as pl
from jax.experimental.pallas import tpu as pltpu
from jax.experimental.pallas import tpu_sc as plsc

NC = 2
NS = 16
NW = NC * NS
CH = 64
GRP = 8
LANES = 16


def _proj_body(x_ref, w_in_ref, b_in_ref, w1_ref, w2_ref, bm_ref, a_ref, b_ref):
    h = jnp.dot(x_ref[...], w_in_ref[...], preferred_element_type=jnp.float32)
    h = jnp.maximum(h + b_in_ref[...], 0.0)
    a_ref[...] = jnp.dot(h, w1_ref[...], preferred_element_type=jnp.float32)
    b_ref[...] = (
        jnp.dot(h, w2_ref[...], preferred_element_type=jnp.float32) + bm_ref[...]
    )


@functools.partial(jax.jit, static_argnames=("blk",))
def _proj(x, w_in, b_in, w1, w2, bm, blk=1000):
    n, d = x.shape
    hid = w_in.shape[1]
    grid = n // blk
    return pl.pallas_call(
        _proj_body,
        grid=(grid,),
        in_specs=[
            pl.BlockSpec((blk, d), lambda i: (i, 0)),
            pl.BlockSpec((d, hid), lambda i: (0, 0)),
            pl.BlockSpec((1, hid), lambda i: (0, 0)),
            pl.BlockSpec((hid, hid), lambda i: (0, 0)),
            pl.BlockSpec((hid, hid), lambda i: (0, 0)),
            pl.BlockSpec((1, hid), lambda i: (0, 0)),
        ],
        out_specs=[
            pl.BlockSpec((blk, hid), lambda i: (i, 0)),
            pl.BlockSpec((blk, hid), lambda i: (i, 0)),
        ],
        out_shape=[
            jax.ShapeDtypeStruct((n, hid), jnp.float32),
            jax.ShapeDtypeStruct((n, hid), jnp.float32),
        ],
    )(x, w_in, b_in.reshape(1, hid), w1, w2, bm.reshape(1, hid))


def _make_sc(nch0, nch1, npad, hid):
    assert nch0 % (2 * GRP) == 0 and nch1 % (2 * GRP) == 0
    rows_per_tile = npad // NS
    assert rows_per_tile % CH == 0
    vpr = hid // LANES

    mesh = plsc.VectorSubcoreMesh(core_axis_name="c", subcore_axis_name="s")

    @functools.partial(
        pl.kernel,
        out_type=jax.ShapeDtypeStruct((NC, npad, hid), jnp.float32),
        mesh=mesh,
        scratch_types=[
            [pltpu.VMEM((GRP, 2 * CH), jnp.int32)] * 2,
            [pltpu.VMEM((GRP, CH), jnp.int32)] * 2,
            [pltpu.VMEM((2 * CH, hid), jnp.float32)] * 2,
            pltpu.VMEM((CH, hid), jnp.float32),
            pltpu.VMEM_SHARED((npad, hid), jnp.float32),
            [pltpu.SemaphoreType.DMA] * 2,
            [pltpu.SemaphoreType.DMA] * 2,
        ],
    )
    def sc_edges(t_hbm, gidx_hbm, darr_hbm, out_hbm,
                 gi, da, abuf, mbuf, acc, sem_i, sem_g):
        cid = lax.axis_index("c")
        sid = lax.axis_index("s")
        chunk0 = jnp.where(cid == 0, sid * nch0, NS * nch0 + sid * nch1)
        ngroups = jnp.where(cid == 0, nch0 // GRP, nch1 // GRP)

        def issue_idx(gp, g):
            row0 = chunk0 + g * GRP
            pltpu.async_copy(gidx_hbm.at[pl.ds(row0, GRP)], gi[gp], sem_i[gp])
            pltpu.async_copy(darr_hbm.at[pl.ds(row0, GRP)], da[gp], sem_i[gp])

        def wait_idx(gp):
            pltpu.make_async_copy(
                gidx_hbm.at[pl.ds(chunk0, GRP)], gi[gp], sem_i[gp]).wait()
            pltpu.make_async_copy(
                darr_hbm.at[pl.ds(chunk0, GRP)], da[gp], sem_i[gp]).wait()

        def issue_gather(p, gp, j):
            pltpu.async_copy(t_hbm.at[gi[gp].at[j]], abuf[p], sem_g[p])

        def wait_gather(p, gp, j):
            pltpu.make_async_copy(t_hbm.at[gi[gp].at[j]], abuf[p], sem_g[p]).wait()

        scope_zero = jax.named_scope("phase_zero")
        scope_zero.__enter__()
        zero = jnp.zeros((LANES,), jnp.float32)

        def _zero_row(r, _):
            for c in range(vpr):
                mbuf[r, pl.ds(c * LANES, LANES)] = zero
            return 0

        lax.fori_loop(0, CH, _zero_row, 0)

        def _zero_acc(i, _):
            pltpu.sync_copy(mbuf, acc.at[pl.ds(sid * rows_per_tile + i * CH, CH)])
            return 0

        lax.fori_loop(0, rows_per_tile // CH, _zero_acc, 0)
        plsc.subcore_barrier()
        scope_zero.__exit__(None, None, None)

        scope_loop = jax.named_scope("phase_loop")
        scope_loop.__enter__()
        issue_idx(0, 0)
        wait_idx(0)
        issue_gather(0, 0, 0)
        issue_gather(1, 0, 1)

        @pl.loop(0, ngroups, step=2)
        def _groups(gbase):
            for gg in range(2):
                g = gbase + gg
                gp = gg
                for j in range(GRP):
                    p = j & 1
                    if j == 0:
                        g_next = jnp.where(g + 1 >= ngroups, 0, g + 1)
                        issue_idx(gp ^ 1, g_next)
                    if j == GRP - 3:
                        wait_idx(gp ^ 1)
                    wait_gather(p, gp, j)

                    def _row(r, _, p=p):
                        for cc in range(vpr):
                            s = pl.ds(cc * LANES, LANES)
                            mbuf[r, s] = jnp.maximum(
                                abuf[p][r, s] + abuf[p][r + CH, s], 0.0)
                        return 0

                    lax.fori_loop(0, CH, _row, 0)
                    if j < GRP - 2:
                        issue_gather(p, gp, j + 2)
                    else:
                        issue_gather(p, gp ^ 1, j + 2 - GRP)
                    pltpu.sync_copy(mbuf, acc.at[da[gp].at[j]], add=True)

        wait_gather(0, 0, 0)
        wait_gather(1, 0, 1)
        plsc.subcore_barrier()
        scope_loop.__exit__(None, None, None)

        with jax.named_scope("phase_drain"):
            r0 = sid * rows_per_tile
            pltpu.sync_copy(acc.at[pl.ds(r0, rows_per_tile)],
                            out_hbm.at[cid, pl.ds(r0, rows_per_tile)])

    return sc_edges


def _merge_body(p_ref, o_ref):
    o_ref[...] = p_ref[0] + p_ref[1]


@functools.partial(jax.jit, static_argnames=("n", "blk"))
def _merge(partials, n, blk=1000):
    npad, hid = partials.shape[1], partials.shape[2]
    return pl.pallas_call(
        _merge_body,
        grid=(n // blk,),
        in_specs=[pl.BlockSpec((2, blk, hid), lambda i: (0, i, 0))],
        out_specs=pl.BlockSpec((blk, hid), lambda i: (i, 0)),
        out_shape=jax.ShapeDtypeStruct((n, hid), jnp.float32),
    )(partials)


def kernel(x, edge_index, W_in, b_in, W_msg, b_msg):
    n, d = x.shape
    hid = W_in.shape[1]
    e = edge_index.shape[1]

    a, b = _proj(x, W_in, b_in, W_msg[:hid], W_msg[hid:], b_msg)

    quantum = NW * CH * 2 * GRP
    e_pad = -(-e // quantum) * quantum
    src = edge_index[0]
    dst = edge_index[1]
    if e_pad > e:
        src = jnp.concatenate([src, jnp.zeros((e_pad - e,), jnp.int32)])
        dst = jnp.concatenate([dst, jnp.full((e_pad - e,), n, jnp.int32)])
    b_pad = jnp.concatenate([b, jnp.zeros((1, hid), jnp.float32)], axis=0)

    table = jnp.concatenate([a, b_pad], axis=0)
    tot_chunks = e_pad // CH
    gidx = jnp.concatenate(
        [src.reshape(tot_chunks, CH), dst.reshape(tot_chunks, CH) + n], axis=1)
    darr = dst.reshape(tot_chunks, CH)

    npad = -(-(n + 1) // (NS * CH)) * (NS * CH)
    tot_pair = e_pad // (CH * NS)
    k0 = int(round(0.8 * tot_pair / (2 * GRP))) * (2 * GRP)
    k1 = tot_pair - k0
    sc_edges = _make_sc(k0, k1, npad, hid)
    partials = sc_edges(table, gidx, darr)

    return _merge(partials, n)


# speedup vs baseline: 1.2269x; 1.0119x over previous
import functools

import jax
import jax.numpy as jnp
from jax import lax
from jax.experimental import pallas as pl
from jax.experimental.pallas import tpu as pltpu
from jax.experimental.pallas import tpu_sc as plsc

NC = 2
NS = 16
NW = NC * NS
CH = 64
GRP = 8
LANES = 16


def _proj_body(x_ref, w_in_ref, b_in_ref, w1_ref, w2_ref, bm_ref, a_ref, b_ref):
    h = jnp.dot(x_ref[...], w_in_ref[...], preferred_element_type=jnp.float32)
    h = jnp.maximum(h + b_in_ref[...], 0.0)
    a_ref[...] = jnp.dot(h, w1_ref[...], preferred_element_type=jnp.float32)
    b_ref[...] = (
        jnp.dot(h, w2_ref[...], preferred_element_type=jnp.float32) + bm_ref[...]
    )


@functools.partial(jax.jit, static_argnames=("blk",))
def _proj(x, w_in, b_in, w1, w2, bm, blk=1000):
    n, d = x.shape
    hid = w_in.shape[1]
    grid = n // blk
    return pl.pallas_call(
        _proj_body,
        grid=(grid,),
        in_specs=[
            pl.BlockSpec((blk, d), lambda i: (i, 0)),
            pl.BlockSpec((d, hid), lambda i: (0, 0)),
            pl.BlockSpec((1, hid), lambda i: (0, 0)),
            pl.BlockSpec((hid, hid), lambda i: (0, 0)),
            pl.BlockSpec((hid, hid), lambda i: (0, 0)),
            pl.BlockSpec((1, hid), lambda i: (0, 0)),
        ],
        out_specs=[
            pl.BlockSpec((blk, hid), lambda i: (i, 0)),
            pl.BlockSpec((blk, hid), lambda i: (i, 0)),
        ],
        out_shape=[
            jax.ShapeDtypeStruct((n, hid), jnp.float32),
            jax.ShapeDtypeStruct((n, hid), jnp.float32),
        ],
    )(x, w_in, b_in.reshape(1, hid), w1, w2, bm.reshape(1, hid))


def _make_sc(nch0, nch1, npad, hid):
    assert nch0 % (2 * GRP) == 0 and nch1 % (2 * GRP) == 0
    rows_per_tile = npad // NS
    assert rows_per_tile % CH == 0
    vpr = hid // LANES

    mesh = plsc.VectorSubcoreMesh(core_axis_name="c", subcore_axis_name="s")

    @functools.partial(
        pl.kernel,
        out_type=jax.ShapeDtypeStruct((NC, npad, hid), jnp.float32),
        mesh=mesh,
        scratch_types=[
            [pltpu.VMEM((GRP, 2 * CH), jnp.int32)] * 2,
            [pltpu.VMEM((GRP, CH), jnp.int32)] * 2,
            [pltpu.VMEM((2 * CH, hid), jnp.float32)] * 2,
            pltpu.VMEM((CH, hid), jnp.float32),
            pltpu.VMEM_SHARED((npad, hid), jnp.float32),
            [pltpu.SemaphoreType.DMA] * 2,
            [pltpu.SemaphoreType.DMA] * 2,
        ],
    )
    def sc_edges(t_hbm, gidx_hbm, darr_hbm, out_hbm,
                 gi, da, abuf, mbuf, acc, sem_i, sem_g):
        cid = lax.axis_index("c")
        sid = lax.axis_index("s")
        chunk0 = jnp.where(cid == 0, sid * nch0, NS * nch0 + sid * nch1)
        ngroups = jnp.where(cid == 0, nch0 // GRP, nch1 // GRP)

        def issue_idx(gp, g):
            row0 = chunk0 + g * GRP
            pltpu.async_copy(gidx_hbm.at[pl.ds(row0, GRP)], gi[gp], sem_i[gp])
            pltpu.async_copy(darr_hbm.at[pl.ds(row0, GRP)], da[gp], sem_i[gp])

        def wait_idx(gp):
            pltpu.make_async_copy(
                gidx_hbm.at[pl.ds(chunk0, GRP)], gi[gp], sem_i[gp]).wait()
            pltpu.make_async_copy(
                darr_hbm.at[pl.ds(chunk0, GRP)], da[gp], sem_i[gp]).wait()

        def issue_gather(p, gp, j):
            pltpu.async_copy(t_hbm.at[gi[gp].at[j]], abuf[p], sem_g[p])

        def wait_gather(p, gp, j):
            pltpu.make_async_copy(t_hbm.at[gi[gp].at[j]], abuf[p], sem_g[p]).wait()

        scope_zero = jax.named_scope("phase_zero")
        scope_zero.__enter__()
        zero = jnp.zeros((LANES,), jnp.float32)

        def _zero_row(r, _):
            for c in range(vpr):
                mbuf[r, pl.ds(c * LANES, LANES)] = zero
            return 0

        lax.fori_loop(0, CH, _zero_row, 0)

        def _zero_acc(i, _):
            pltpu.sync_copy(mbuf, acc.at[pl.ds(sid * rows_per_tile + i * CH, CH)])
            return 0

        lax.fori_loop(0, rows_per_tile // CH, _zero_acc, 0)
        plsc.subcore_barrier()
        scope_zero.__exit__(None, None, None)

        scope_loop = jax.named_scope("phase_loop")
        scope_loop.__enter__()
        issue_idx(0, 0)
        wait_idx(0)
        issue_gather(0, 0, 0)
        issue_gather(1, 0, 1)

        @pl.loop(0, ngroups, step=2)
        def _groups(gbase):
            for gg in range(2):
                g = gbase + gg
                gp = gg
                for j in range(GRP):
                    p = j & 1
                    if j == 0:
                        g_next = jnp.where(g + 1 >= ngroups, 0, g + 1)
                        issue_idx(gp ^ 1, g_next)
                    if j == GRP - 3:
                        wait_idx(gp ^ 1)
                    wait_gather(p, gp, j)

                    def _row(r, _, p=p):
                        for cc in range(vpr):
                            s = pl.ds(cc * LANES, LANES)
                            mbuf[r, s] = jnp.maximum(
                                abuf[p][r, s] + abuf[p][r + CH, s], 0.0)
                        return 0

                    lax.fori_loop(0, CH, _row, 0)
                    if j < GRP - 2:
                        issue_gather(p, gp, j + 2)
                    else:
                        issue_gather(p, gp ^ 1, j + 2 - GRP)
                    pltpu.sync_copy(mbuf, acc.at[da[gp].at[j]], add=True)

        wait_gather(0, 0, 0)
        wait_gather(1, 0, 1)
        plsc.subcore_barrier()
        scope_loop.__exit__(None, None, None)

        with jax.named_scope("phase_drain"):
            r0 = sid * rows_per_tile
            pltpu.sync_copy(acc.at[pl.ds(r0, rows_per_tile)],
                            out_hbm.at[cid, pl.ds(r0, rows_per_tile)])

    return sc_edges


def _merge_body(p_ref, o_ref):
    o_ref[...] = p_ref[0] + p_ref[1]


@functools.partial(jax.jit, static_argnames=("n", "blk"))
def _merge(partials, n, blk=1000):
    npad, hid = partials.shape[1], partials.shape[2]
    return pl.pallas_call(
        _merge_body,
        grid=(n // blk,),
        in_specs=[pl.BlockSpec((2, blk, hid), lambda i: (0, i, 0))],
        out_specs=pl.BlockSpec((blk, hid), lambda i: (i, 0)),
        out_shape=jax.ShapeDtypeStruct((n, hid), jnp.float32),
    )(partials)


def kernel(x, edge_index, W_in, b_in, W_msg, b_msg):
    n, d = x.shape
    hid = W_in.shape[1]
    e = edge_index.shape[1]

    a, b = _proj(x, W_in, b_in, W_msg[:hid], W_msg[hid:], b_msg)

    quantum = NW * CH * 2 * GRP
    e_pad = -(-e // quantum) * quantum
    src = edge_index[0]
    dst = edge_index[1]
    if e_pad > e:
        src = jnp.concatenate([src, jnp.zeros((e_pad - e,), jnp.int32)])
        dst = jnp.concatenate([dst, jnp.full((e_pad - e,), n, jnp.int32)])
    b_pad = jnp.concatenate([b, jnp.zeros((1, hid), jnp.float32)], axis=0)

    table = jnp.concatenate([a, b_pad], axis=0)
    tot_chunks = e_pad // CH
    gidx = jnp.concatenate(
        [src.reshape(tot_chunks, CH), dst.reshape(tot_chunks, CH) + n], axis=1)
    darr = dst.reshape(tot_chunks, CH)

    npad = -(-(n + 1) // (NS * CH)) * (NS * CH)
    tot_pair = e_pad // (CH * NS)
    k0 = int(round(0.85 * tot_pair / (2 * GRP))) * (2 * GRP)
    k1 = tot_pair - k0
    sc_edges = _make_sc(k0, k1, npad, hid)
    partials = sc_edges(table, gidx, darr)

    return _merge(partials, n)


# speedup vs baseline: 1.2491x; 1.0182x over previous
import functools

import jax
import jax.numpy as jnp
from jax import lax
from jax.experimental import pallas as pl
from jax.experimental.pallas import tpu as pltpu
from jax.experimental.pallas import tpu_sc as plsc

NC = 2
NS = 16
NW = NC * NS
CH = 64
GRP = 8
LANES = 16


def _proj_body(x_ref, w_in_ref, b_in_ref, w1_ref, w2_ref, bm_ref, a_ref, b_ref):
    h = jnp.dot(x_ref[...], w_in_ref[...], preferred_element_type=jnp.float32)
    h = jnp.maximum(h + b_in_ref[...], 0.0)
    a_ref[...] = jnp.dot(h, w1_ref[...], preferred_element_type=jnp.float32)
    b_ref[...] = (
        jnp.dot(h, w2_ref[...], preferred_element_type=jnp.float32) + bm_ref[...]
    )


@functools.partial(jax.jit, static_argnames=("blk",))
def _proj(x, w_in, b_in, w1, w2, bm, blk=1000):
    n, d = x.shape
    hid = w_in.shape[1]
    grid = n // blk
    return pl.pallas_call(
        _proj_body,
        grid=(grid,),
        in_specs=[
            pl.BlockSpec((blk, d), lambda i: (i, 0)),
            pl.BlockSpec((d, hid), lambda i: (0, 0)),
            pl.BlockSpec((1, hid), lambda i: (0, 0)),
            pl.BlockSpec((hid, hid), lambda i: (0, 0)),
            pl.BlockSpec((hid, hid), lambda i: (0, 0)),
            pl.BlockSpec((1, hid), lambda i: (0, 0)),
        ],
        out_specs=[
            pl.BlockSpec((blk, hid), lambda i: (i, 0)),
            pl.BlockSpec((blk, hid), lambda i: (i, 0)),
        ],
        out_shape=[
            jax.ShapeDtypeStruct((n, hid), jnp.float32),
            jax.ShapeDtypeStruct((n, hid), jnp.float32),
        ],
    )(x, w_in, b_in.reshape(1, hid), w1, w2, bm.reshape(1, hid))


def _make_sc(nch0, nch1, npad, hid):
    assert nch0 % (2 * GRP) == 0 and nch1 % (2 * GRP) == 0
    rows_per_tile = npad // NS
    assert rows_per_tile % CH == 0
    vpr = hid // LANES

    mesh = plsc.VectorSubcoreMesh(core_axis_name="c", subcore_axis_name="s")

    @functools.partial(
        pl.kernel,
        out_type=jax.ShapeDtypeStruct((NC, npad, hid), jnp.float32),
        mesh=mesh,
        scratch_types=[
            [pltpu.VMEM((GRP, 2 * CH), jnp.int32)] * 2,
            [pltpu.VMEM((GRP, CH), jnp.int32)] * 2,
            [pltpu.VMEM((2 * CH, hid), jnp.float32)] * 2,
            pltpu.VMEM((CH, hid), jnp.float32),
            pltpu.VMEM_SHARED((npad, hid), jnp.float32),
            [pltpu.SemaphoreType.DMA] * 2,
            [pltpu.SemaphoreType.DMA] * 2,
        ],
    )
    def sc_edges(t_hbm, gidx_hbm, darr_hbm, out_hbm,
                 gi, da, abuf, mbuf, acc, sem_i, sem_g):
        cid = lax.axis_index("c")
        sid = lax.axis_index("s")
        chunk0 = jnp.where(cid == 0, sid * nch0, NS * nch0 + sid * nch1)
        ngroups = jnp.where(cid == 0, nch0 // GRP, nch1 // GRP)

        def issue_idx(gp, g):
            row0 = chunk0 + g * GRP
            pltpu.async_copy(gidx_hbm.at[pl.ds(row0, GRP)], gi[gp], sem_i[gp])
            pltpu.async_copy(darr_hbm.at[pl.ds(row0, GRP)], da[gp], sem_i[gp])

        def wait_idx(gp):
            pltpu.make_async_copy(
                gidx_hbm.at[pl.ds(chunk0, GRP)], gi[gp], sem_i[gp]).wait()
            pltpu.make_async_copy(
                darr_hbm.at[pl.ds(chunk0, GRP)], da[gp], sem_i[gp]).wait()

        def issue_gather(p, gp, j):
            pltpu.async_copy(t_hbm.at[gi[gp].at[j]], abuf[p], sem_g[p])

        def wait_gather(p, gp, j):
            pltpu.make_async_copy(t_hbm.at[gi[gp].at[j]], abuf[p], sem_g[p]).wait()

        scope_zero = jax.named_scope("phase_zero")
        scope_zero.__enter__()
        zero = jnp.zeros((LANES,), jnp.float32)

        def _zero_row(r, _):
            for c in range(vpr):
                mbuf[r, pl.ds(c * LANES, LANES)] = zero
            return 0

        lax.fori_loop(0, CH, _zero_row, 0)

        def _zero_acc(i, _):
            pltpu.sync_copy(mbuf, acc.at[pl.ds(sid * rows_per_tile + i * CH, CH)])
            return 0

        lax.fori_loop(0, rows_per_tile // CH, _zero_acc, 0)
        plsc.subcore_barrier()
        scope_zero.__exit__(None, None, None)

        scope_loop = jax.named_scope("phase_loop")
        scope_loop.__enter__()
        issue_idx(0, 0)
        wait_idx(0)
        issue_gather(0, 0, 0)
        issue_gather(1, 0, 1)

        @pl.loop(0, ngroups, step=2)
        def _groups(gbase):
            for gg in range(2):
                g = gbase + gg
                gp = gg
                for j in range(GRP):
                    p = j & 1
                    if j == 0:
                        g_next = jnp.where(g + 1 >= ngroups, 0, g + 1)
                        issue_idx(gp ^ 1, g_next)
                    if j == GRP - 3:
                        wait_idx(gp ^ 1)
                    wait_gather(p, gp, j)

                    def _row(r, _, p=p):
                        for cc in range(vpr):
                            s = pl.ds(cc * LANES, LANES)
                            mbuf[r, s] = jnp.maximum(
                                abuf[p][r, s] + abuf[p][r + CH, s], 0.0)
                        return 0

                    lax.fori_loop(0, CH, _row, 0)
                    if j < GRP - 2:
                        issue_gather(p, gp, j + 2)
                    else:
                        issue_gather(p, gp ^ 1, j + 2 - GRP)
                    pltpu.sync_copy(mbuf, acc.at[da[gp].at[j]], add=True)

        wait_gather(0, 0, 0)
        wait_gather(1, 0, 1)
        plsc.subcore_barrier()
        scope_loop.__exit__(None, None, None)

        with jax.named_scope("phase_drain"):
            r0 = sid * rows_per_tile
            pltpu.sync_copy(acc.at[pl.ds(r0, rows_per_tile)],
                            out_hbm.at[cid, pl.ds(r0, rows_per_tile)])

    return sc_edges


def _merge_body(p_ref, o_ref):
    o_ref[...] = p_ref[0] + p_ref[1]


@functools.partial(jax.jit, static_argnames=("n", "blk"))
def _merge(partials, n, blk=1000):
    npad, hid = partials.shape[1], partials.shape[2]
    return pl.pallas_call(
        _merge_body,
        grid=(n // blk,),
        in_specs=[pl.BlockSpec((2, blk, hid), lambda i: (0, i, 0))],
        out_specs=pl.BlockSpec((blk, hid), lambda i: (i, 0)),
        out_shape=jax.ShapeDtypeStruct((n, hid), jnp.float32),
    )(partials)


def kernel(x, edge_index, W_in, b_in, W_msg, b_msg):
    n, d = x.shape
    hid = W_in.shape[1]
    e = edge_index.shape[1]

    a, b = _proj(x, W_in, b_in, W_msg[:hid], W_msg[hid:], b_msg)

    quantum = NW * CH * 2 * GRP
    e_pad = -(-e // quantum) * quantum
    src = edge_index[0]
    dst = edge_index[1]
    if e_pad > e:
        src = jnp.concatenate([src, jnp.zeros((e_pad - e,), jnp.int32)])
        dst = jnp.concatenate([dst, jnp.full((e_pad - e,), n, jnp.int32)])
    b_pad = jnp.concatenate([b, jnp.zeros((1, hid), jnp.float32)], axis=0)

    table = jnp.concatenate([a, b_pad], axis=0)
    tot_chunks = e_pad // CH
    gidx = jnp.concatenate(
        [src.reshape(tot_chunks, CH), dst.reshape(tot_chunks, CH) + n], axis=1)
    darr = dst.reshape(tot_chunks, CH)

    npad = -(-(n + 1) // (NS * CH)) * (NS * CH)
    tot_pair = e_pad // (CH * NS)
    k0 = int(round(0.9 * tot_pair / (2 * GRP))) * (2 * GRP)
    k1 = tot_pair - k0
    sc_edges = _make_sc(k0, k1, npad, hid)
    partials = sc_edges(table, gidx, darr)

    return _merge(partials, n)
